# baseline (speedup 1.0000x reference)
import functools

import jax
import jax.numpy as jnp
from jax import lax
from jax.experimental import pallas as pl
from jax.experimental.pallas import tpu as pltpu
from jax.experimental.pallas import tpu_sc as plsc

N = 10000
E = 320000
D = 128
HID = 2 * D
EDGE_DIM = 7

NC = 2
NS = 16
NW = NC * NS
EPW = E // NW
CH = 80
NCH = EPW // CH
N_PAD = 10240
ZROWS = N_PAD // NS
OROWS = 624
OTAIL = N - NS * OROWS

BE = 3200


def _edge_body(*refs, relu, layer):
    if relu:
        ea_ref, w_ref, b_ref, c_ref, o_ref = refs
        bias = b_ref[layer:layer + 1, :] + c_ref[...]
    else:
        ea_ref, w_ref, b_ref, o_ref = refs
        bias = b_ref[layer:layer + 1, :]
    y = jnp.broadcast_to(bias, (BE, D))
    for k in range(EDGE_DIM):
        y = y + ea_ref[:, k:k + 1] * w_ref[0, k:k + 1, :]
    if relu:
        y = jnp.maximum(y, 0.0)
    o_ref[...] = y


def _edge_stage(layer, relu, interpret=False):
    in_specs = [
        pl.BlockSpec((BE, EDGE_DIM), lambda i: (i, 0)),
        pl.BlockSpec((1, EDGE_DIM, D), lambda i: (layer, 0, 0)),
        pl.BlockSpec((2, D), lambda i: (0, 0)),
    ]
    if relu:
        in_specs.append(pl.BlockSpec((1, D), lambda i: (0, 0)))
    return pl.pallas_call(
        functools.partial(_edge_body, relu=relu, layer=layer),
        grid=(E // BE,),
        in_specs=in_specs,
        out_specs=pl.BlockSpec((BE, D), lambda i: (i, 0)),
        out_shape=jax.ShapeDtypeStruct((E, D), jnp.float32),
        interpret=interpret,
    )


RING = 4
AHEAD = 2


def _make_sc_scatter(interpret=False):
    scratch = (
        [pltpu.VMEM((CH,), jnp.int32) for _ in range(RING)]
        + [pltpu.VMEM((CH, D), jnp.float32) for _ in range(RING)]
        + [pltpu.VMEM_SHARED((N_PAD, D), jnp.float32)]
        + [pltpu.SemaphoreType.DMA for _ in range(3 * RING)]
    )
    mesh = plsc.VectorSubcoreMesh(core_axis_name="c", subcore_axis_name="s")

    @functools.partial(
        pl.kernel,
        out_type=jax.ShapeDtypeStruct((NC, N, D), jnp.float32),
        mesh=mesh,
        scratch_types=scratch,
        interpret=interpret,
    )
    def sc_stage(emb_hbm, ei_hbm, z_hbm, out_hbm, *rest):
        dsti = rest[:RING]
        ebuf = rest[RING:2 * RING]
        agg_sh = rest[2 * RING]
        dsem = rest[2 * RING + 1:3 * RING + 1]
        esem = rest[3 * RING + 1:4 * RING + 1]
        ssem = rest[4 * RING + 1:]

        cid = lax.axis_index("c")
        sid = lax.axis_index("s")
        wid = cid * NS + sid
        base = wid * EPW

        def load(j, u):
            eb = base + j * CH
            pltpu.async_copy(ei_hbm.at[pl.ds(E + eb, CH)], dsti[u], dsem[u])
            pltpu.async_copy(emb_hbm.at[pl.ds(eb, CH)], ebuf[u], esem[u])

        def wait_load(u):
            pltpu.make_async_copy(
                ei_hbm.at[pl.ds(0, CH)], dsti[u], dsem[u]).wait()
            pltpu.make_async_copy(
                emb_hbm.at[pl.ds(0, CH)], ebuf[u], esem[u]).wait()

        def scat(u):
            pltpu.async_copy(ebuf[u], agg_sh.at[dsti[u]], ssem[u], add=True)

        def wait_scat(u):
            pltpu.make_async_copy(
                ebuf[u], agg_sh.at[dsti[u]], ssem[u]).wait()

        pltpu.sync_copy(z_hbm, agg_sh.at[pl.ds(sid * ZROWS, ZROWS)])
        plsc.subcore_barrier()

        for j in range(AHEAD):
            load(j, j)
        for j in range(RING):
            wait_load(j)
            scat(j)
            if j >= AHEAD:
                wait_scat(j - AHEAD)
            load(j + AHEAD, (j + AHEAD) % RING)

        loop_end = ((NCH - AHEAD) // RING) * RING

        @pl.loop(RING, loop_end, step=RING)
        def _(jb):
            for u in range(RING):
                j = jb + u
                wait_load(u)
                scat(u)
                wait_scat((u + AHEAD) % RING)
                load(j + AHEAD, (u + AHEAD) % RING)

        for j in range(loop_end, NCH):
            u = j % RING
            wait_load(u)
            scat(u)
            wait_scat((u + AHEAD) % RING)
            if j + AHEAD < NCH:
                load(j + AHEAD, (j + AHEAD) % RING)
        for j in range(NCH - AHEAD, NCH):
            wait_scat(j % RING)

        plsc.subcore_barrier()
        pltpu.sync_copy(
            agg_sh.at[pl.ds(sid * OROWS, OROWS)],
            out_hbm.at[cid, pl.ds(sid * OROWS, OROWS)],
        )

        @pl.when(sid == 0)
        def _():
            pltpu.sync_copy(
                agg_sh.at[pl.ds(NS * OROWS, OTAIL)],
                out_hbm.at[cid, pl.ds(NS * OROWS, OTAIL)],
            )

    return sc_stage


def _make_sc_stage(with_gather, interpret=False):
    scratch = [
        pltpu.VMEM((CH,), jnp.int32),
        pltpu.VMEM((CH,), jnp.int32),
        pltpu.VMEM((CH, D), jnp.float32),
        pltpu.VMEM((CH, D), jnp.float32),
        pltpu.VMEM_SHARED((N_PAD, D), jnp.float32),
        pltpu.SemaphoreType.DMA,
        pltpu.SemaphoreType.DMA,
        pltpu.SemaphoreType.DMA,
        pltpu.SemaphoreType.DMA,
    ]
    if with_gather:
        scratch += [
            pltpu.VMEM((CH,), jnp.int32),
            pltpu.VMEM((CH,), jnp.int32),
            pltpu.VMEM((CH, D), jnp.float32),
            pltpu.VMEM((CH, D), jnp.float32),
            pltpu.SemaphoreType.DMA,
            pltpu.SemaphoreType.DMA,
            pltpu.SemaphoreType.DMA,
            pltpu.SemaphoreType.DMA,
        ]
    mesh = plsc.VectorSubcoreMesh(core_axis_name="c", subcore_axis_name="s")

    @functools.partial(
        pl.kernel,
        out_type=jax.ShapeDtypeStruct((NC, N, D), jnp.float32),
        mesh=mesh,
        scratch_types=scratch,
        interpret=interpret,
    )
    def sc_stage(*args):
        if with_gather:
            (emb_hbm, ei_hbm, z_hbm, h_hbm, out_hbm,
             dsti0, dsti1, ebuf0, ebuf1, agg_sh, sd0, sd1, se0, se1,
             srci0, srci1, rows0, rows1, ss0, ss1, sg0, sg1) = args
            srci = (srci0, srci1)
            rows = (rows0, rows1)
            ssem = (ss0, ss1)
            gsem = (sg0, sg1)
        else:
            (emb_hbm, ei_hbm, z_hbm, out_hbm,
             dsti0, dsti1, ebuf0, ebuf1, agg_sh, sd0, sd1, se0, se1) = args
        dsti = (dsti0, dsti1)
        ebuf = (ebuf0, ebuf1)
        dsem = (sd0, sd1)
        esem = (se0, se1)

        cid = lax.axis_index("c")
        sid = lax.axis_index("s")
        wid = cid * NS + sid
        base = wid * EPW

        def start_loads(j, s):
            eb = base + j * CH
            pltpu.make_async_copy(
                ei_hbm.at[pl.ds(E + eb, CH)], dsti[s], dsem[s]).start()
            pltpu.make_async_copy(
                emb_hbm.at[pl.ds(eb, CH)], ebuf[s], esem[s]).start()
            if with_gather:
                pltpu.make_async_copy(
                    ei_hbm.at[pl.ds(eb, CH)], srci[s], ssem[s]).start()

        def wait_loads(s):
            pltpu.make_async_copy(
                ei_hbm.at[pl.ds(0, CH)], dsti[s], dsem[s]).wait()
            pltpu.make_async_copy(
                emb_hbm.at[pl.ds(0, CH)], ebuf[s], esem[s]).wait()

        def start_gather(s):
            pltpu.make_async_copy(
                ei_hbm.at[pl.ds(0, CH)], srci[s], ssem[s]).wait()
            pltpu.make_async_copy(h_hbm.at[srci[s]], rows[s], gsem[s]).start()

        def process(s):
            wait_loads(s)
            if with_gather:
                pltpu.make_async_copy(
                    h_hbm.at[srci[s]], rows[s], gsem[s]).wait()

                @pl.loop(0, CH, step=2)
                def _(r):
                    for rr in range(2):
                        for q in range(D // 16):
                            sl = pl.ds(q * 16, 16)
                            v = (ebuf[s].at[r + rr, sl][...]
                                 + rows[s].at[r + rr, sl][...])
                            ebuf[s].at[r + rr, sl][...] = jnp.maximum(v, 0.0)

                pltpu.sync_copy(ebuf[s], agg_sh.at[dsti[s]], add=True)
            else:
                pltpu.sync_copy(ebuf[s], agg_sh.at[dsti[s]], add=True)

        pltpu.sync_copy(z_hbm, agg_sh.at[pl.ds(sid * ZROWS, ZROWS)])
        plsc.subcore_barrier()

        start_loads(0, 0)
        start_loads(1, 1)
        if with_gather:
            start_gather(0)

        @pl.loop(0, (NCH - 1) // 2)
        def _(k):
            j = 2 * k
            process(0)
            start_loads(j + 2, 0)
            if with_gather:
                start_gather(1)
            process(1)

            @pl.when(j + 3 < NCH)
            def _():
                start_loads(j + 3, 1)

            if with_gather:
                start_gather(0)

        process(0)

        plsc.subcore_barrier()
        pltpu.sync_copy(
            agg_sh.at[pl.ds(sid * OROWS, OROWS)],
            out_hbm.at[cid, pl.ds(sid * OROWS, OROWS)],
        )

        @pl.when(sid == 0)
        def _():
            pltpu.sync_copy(
                agg_sh.at[pl.ds(NS * OROWS, OTAIL)],
                out_hbm.at[cid, pl.ds(NS * OROWS, OTAIL)],
            )

    return sc_stage


def _node_body(h_ref, a_ref, e_ref, w1_ref, b1_ref, g1_ref, be1_ref,
               w2_ref, b2_ref, go_ref, bo_ref, o_ref, *, final_relu, layer):
    sl = slice(layer, layer + 1)
    agg = a_ref[0] + a_ref[1]
    z = (1.0 + e_ref[0, layer]) * h_ref[...] + agg
    t = jnp.dot(z, w1_ref[0], preferred_element_type=jnp.float32)
    t = t + b1_ref[sl, :]
    m = jnp.mean(t, axis=0, keepdims=True)
    v = jnp.mean((t - m) ** 2, axis=0, keepdims=True)
    t = (t - m) * lax.rsqrt(v + 1e-5) * g1_ref[sl, :] + be1_ref[sl, :]
    t = jnp.maximum(t, 0.0)
    u = jnp.dot(t, w2_ref[0], preferred_element_type=jnp.float32)
    u = u + b2_ref[sl, :]
    m2 = jnp.mean(u, axis=0, keepdims=True)
    v2 = jnp.mean((u - m2) ** 2, axis=0, keepdims=True)
    y = (u - m2) * lax.rsqrt(v2 + 1e-5) * go_ref[sl, :] + bo_ref[sl, :]
    if final_relu:
        y = jnp.maximum(y, 0.0)
    o_ref[...] = y


def _node_stage(layer, final_relu, h_rows, interpret=False):
    l = layer
    return pl.pallas_call(
        functools.partial(_node_body, final_relu=final_relu, layer=l),
        grid=(1,),
        in_specs=[
            pl.BlockSpec((h_rows, D), lambda i: (0, 0)),
            pl.BlockSpec((NC, N, D), lambda i: (0, 0, 0)),
            pl.BlockSpec((1, 2), lambda i: (0, 0)),
            pl.BlockSpec((1, D, HID), lambda i: (l, 0, 0)),
            pl.BlockSpec((2, HID), lambda i: (0, 0)),
            pl.BlockSpec((2, HID), lambda i: (0, 0)),
            pl.BlockSpec((2, HID), lambda i: (0, 0)),
            pl.BlockSpec((1, HID, D), lambda i: (l, 0, 0)),
            pl.BlockSpec((2, D), lambda i: (0, 0)),
            pl.BlockSpec((2, D), lambda i: (0, 0)),
            pl.BlockSpec((2, D), lambda i: (0, 0)),
        ],
        out_specs=pl.BlockSpec((N, D), lambda i: (0, 0)),
        out_shape=jax.ShapeDtypeStruct((N, D), jnp.float32),
        interpret=interpret,
    )


def kernel(x, edge_index, edge_attr, node_table, edge_W, edge_b, eps,
           W1, b1, g1, be1, W2, b2, g_out, b_out):
    f32 = jnp.float32
    ei = edge_index.reshape(2 * E)
    eps2 = eps.reshape(1, 2)

    msg0 = _edge_stage(0, relu=True)(edge_attr, edge_W, edge_b, node_table)
    e1 = _edge_stage(1, relu=False)(edge_attr, edge_W, edge_b)

    zeros = jnp.zeros((ZROWS, D), f32)
    agg0 = _make_sc_scatter()(msg0, ei, zeros)

    h1 = _node_stage(0, final_relu=True, h_rows=1)(
        node_table, agg0, eps2, W1, b1, g1, be1, W2, b2, g_out, b_out)

    agg1 = _make_sc_stage(with_gather=True)(e1, ei, zeros, h1)

    h2 = _node_stage(1, final_relu=False, h_rows=N)(
        h1, agg1, eps2, W1, b1, g1, be1, W2, b2, g_out, b_out)
    return h2


# speedup vs baseline: 1.4559x; 1.4559x over previous
import functools

import jax
import jax.numpy as jnp
from jax import lax
from jax.experimental import pallas as pl
from jax.experimental.pallas import tpu as pltpu
from jax.experimental.pallas import tpu_sc as plsc

N = 10000
E = 320000
D = 128
HID = 2 * D
EDGE_DIM = 7

NC = 2
NS = 16
NW = NC * NS
EPW = E // NW
CH = 80
NCH = EPW // CH
N_PAD = 10240
ZROWS = N_PAD // NS
OROWS = 624
OTAIL = N - NS * OROWS

BE = 3200


def _edge_body(*refs, relu, layer):
    if relu:
        ea_ref, w_ref, b_ref, c_ref, o_ref = refs
        bias = b_ref[layer:layer + 1, :] + c_ref[...]
    else:
        ea_ref, w_ref, b_ref, o_ref = refs
        bias = b_ref[layer:layer + 1, :]
    y = jnp.dot(ea_ref[...], w_ref[0], preferred_element_type=jnp.float32)
    y = y + bias
    if relu:
        y = jnp.maximum(y, 0.0)
    o_ref[...] = y


def _edge_stage(layer, relu, interpret=False):
    in_specs = [
        pl.BlockSpec((BE, EDGE_DIM), lambda i: (i, 0)),
        pl.BlockSpec((1, EDGE_DIM, D), lambda i: (layer, 0, 0)),
        pl.BlockSpec((2, D), lambda i: (0, 0)),
    ]
    if relu:
        in_specs.append(pl.BlockSpec((1, D), lambda i: (0, 0)))
    return pl.pallas_call(
        functools.partial(_edge_body, relu=relu, layer=layer),
        grid=(E // BE,),
        in_specs=in_specs,
        out_specs=pl.BlockSpec((BE, D), lambda i: (i, 0)),
        out_shape=jax.ShapeDtypeStruct((E, D), jnp.float32),
        interpret=interpret,
    )


RING = 4
AHEAD = 2


def _make_sc_scatter(interpret=False):
    scratch = (
        [pltpu.VMEM((CH,), jnp.int32) for _ in range(RING)]
        + [pltpu.VMEM((CH, D), jnp.float32) for _ in range(RING)]
        + [pltpu.VMEM_SHARED((N_PAD, D), jnp.float32)]
        + [pltpu.SemaphoreType.DMA for _ in range(3 * RING)]
    )
    mesh = plsc.VectorSubcoreMesh(core_axis_name="c", subcore_axis_name="s")

    @functools.partial(
        pl.kernel,
        out_type=jax.ShapeDtypeStruct((NC, N, D), jnp.float32),
        mesh=mesh,
        scratch_types=scratch,
        interpret=interpret,
    )
    def sc_stage(emb_hbm, ei_hbm, z_hbm, out_hbm, *rest):
        dsti = rest[:RING]
        ebuf = rest[RING:2 * RING]
        agg_sh = rest[2 * RING]
        dsem = rest[2 * RING + 1:3 * RING + 1]
        esem = rest[3 * RING + 1:4 * RING + 1]
        ssem = rest[4 * RING + 1:]

        cid = lax.axis_index("c")
        sid = lax.axis_index("s")
        wid = cid * NS + sid
        base = wid * EPW

        def load(j, u):
            eb = base + j * CH
            pltpu.async_copy(ei_hbm.at[pl.ds(E + eb, CH)], dsti[u], dsem[u])
            pltpu.async_copy(emb_hbm.at[pl.ds(eb, CH)], ebuf[u], esem[u])

        def wait_load(u):
            pltpu.make_async_copy(
                ei_hbm.at[pl.ds(0, CH)], dsti[u], dsem[u]).wait()
            pltpu.make_async_copy(
                emb_hbm.at[pl.ds(0, CH)], ebuf[u], esem[u]).wait()

        def scat(u):
            pltpu.async_copy(ebuf[u], agg_sh.at[dsti[u]], ssem[u], add=True)

        def wait_scat(u):
            pltpu.make_async_copy(
                ebuf[u], agg_sh.at[dsti[u]], ssem[u]).wait()

        pltpu.sync_copy(z_hbm, agg_sh.at[pl.ds(sid * ZROWS, ZROWS)])
        plsc.subcore_barrier()

        for j in range(AHEAD):
            load(j, j)
        for j in range(RING):
            wait_load(j)
            scat(j)
            if j >= AHEAD:
                wait_scat(j - AHEAD)
            load(j + AHEAD, (j + AHEAD) % RING)

        loop_end = ((NCH - AHEAD) // RING) * RING

        @pl.loop(RING, loop_end, step=RING)
        def _(jb):
            for u in range(RING):
                j = jb + u
                wait_load(u)
                scat(u)
                wait_scat((u + AHEAD) % RING)
                load(j + AHEAD, (u + AHEAD) % RING)

        for j in range(loop_end, NCH):
            u = j % RING
            wait_load(u)
            scat(u)
            wait_scat((u + AHEAD) % RING)
            if j + AHEAD < NCH:
                load(j + AHEAD, (j + AHEAD) % RING)
        for j in range(NCH - AHEAD, NCH):
            wait_scat(j % RING)

        plsc.subcore_barrier()
        pltpu.sync_copy(
            agg_sh.at[pl.ds(sid * OROWS, OROWS)],
            out_hbm.at[cid, pl.ds(sid * OROWS, OROWS)],
        )

        @pl.when(sid == 0)
        def _():
            pltpu.sync_copy(
                agg_sh.at[pl.ds(NS * OROWS, OTAIL)],
                out_hbm.at[cid, pl.ds(NS * OROWS, OTAIL)],
            )

    return sc_stage


def _make_sc_stage(with_gather, interpret=False):
    scratch = [
        pltpu.VMEM((CH,), jnp.int32),
        pltpu.VMEM((CH,), jnp.int32),
        pltpu.VMEM((CH, D), jnp.float32),
        pltpu.VMEM((CH, D), jnp.float32),
        pltpu.VMEM_SHARED((N_PAD, D), jnp.float32),
        pltpu.SemaphoreType.DMA,
        pltpu.SemaphoreType.DMA,
        pltpu.SemaphoreType.DMA,
        pltpu.SemaphoreType.DMA,
    ]
    if with_gather:
        scratch += [
            pltpu.VMEM((CH,), jnp.int32),
            pltpu.VMEM((CH,), jnp.int32),
            pltpu.VMEM((CH, D), jnp.float32),
            pltpu.VMEM((CH, D), jnp.float32),
            pltpu.SemaphoreType.DMA,
            pltpu.SemaphoreType.DMA,
            pltpu.SemaphoreType.DMA,
            pltpu.SemaphoreType.DMA,
        ]
    mesh = plsc.VectorSubcoreMesh(core_axis_name="c", subcore_axis_name="s")

    @functools.partial(
        pl.kernel,
        out_type=jax.ShapeDtypeStruct((NC, N, D), jnp.float32),
        mesh=mesh,
        scratch_types=scratch,
        interpret=interpret,
    )
    def sc_stage(*args):
        if with_gather:
            (emb_hbm, ei_hbm, z_hbm, h_hbm, out_hbm,
             dsti0, dsti1, ebuf0, ebuf1, agg_sh, sd0, sd1, se0, se1,
             srci0, srci1, rows0, rows1, ss0, ss1, sg0, sg1) = args
            srci = (srci0, srci1)
            rows = (rows0, rows1)
            ssem = (ss0, ss1)
            gsem = (sg0, sg1)
        else:
            (emb_hbm, ei_hbm, z_hbm, out_hbm,
             dsti0, dsti1, ebuf0, ebuf1, agg_sh, sd0, sd1, se0, se1) = args
        dsti = (dsti0, dsti1)
        ebuf = (ebuf0, ebuf1)
        dsem = (sd0, sd1)
        esem = (se0, se1)

        cid = lax.axis_index("c")
        sid = lax.axis_index("s")
        wid = cid * NS + sid
        base = wid * EPW

        def start_loads(j, s):
            eb = base + j * CH
            pltpu.make_async_copy(
                ei_hbm.at[pl.ds(E + eb, CH)], dsti[s], dsem[s]).start()
            pltpu.make_async_copy(
                emb_hbm.at[pl.ds(eb, CH)], ebuf[s], esem[s]).start()
            if with_gather:
                pltpu.make_async_copy(
                    ei_hbm.at[pl.ds(eb, CH)], srci[s], ssem[s]).start()

        def wait_loads(s):
            pltpu.make_async_copy(
                ei_hbm.at[pl.ds(0, CH)], dsti[s], dsem[s]).wait()
            pltpu.make_async_copy(
                emb_hbm.at[pl.ds(0, CH)], ebuf[s], esem[s]).wait()

        def start_gather(s):
            pltpu.make_async_copy(
                ei_hbm.at[pl.ds(0, CH)], srci[s], ssem[s]).wait()
            pltpu.make_async_copy(h_hbm.at[srci[s]], rows[s], gsem[s]).start()

        def process(s):
            wait_loads(s)
            if with_gather:
                pltpu.make_async_copy(
                    h_hbm.at[srci[s]], rows[s], gsem[s]).wait()

                @pl.loop(0, CH, step=2)
                def _(r):
                    for rr in range(2):
                        for q in range(D // 16):
                            sl = pl.ds(q * 16, 16)
                            v = (ebuf[s].at[r + rr, sl][...]
                                 + rows[s].at[r + rr, sl][...])
                            ebuf[s].at[r + rr, sl][...] = jnp.maximum(v, 0.0)

                pltpu.sync_copy(ebuf[s], agg_sh.at[dsti[s]], add=True)
            else:
                pltpu.sync_copy(ebuf[s], agg_sh.at[dsti[s]], add=True)

        pltpu.sync_copy(z_hbm, agg_sh.at[pl.ds(sid * ZROWS, ZROWS)])
        plsc.subcore_barrier()

        start_loads(0, 0)
        start_loads(1, 1)
        if with_gather:
            start_gather(0)

        @pl.loop(0, (NCH - 1) // 2)
        def _(k):
            j = 2 * k
            process(0)
            start_loads(j + 2, 0)
            if with_gather:
                start_gather(1)
            process(1)

            @pl.when(j + 3 < NCH)
            def _():
                start_loads(j + 3, 1)

            if with_gather:
                start_gather(0)

        process(0)

        plsc.subcore_barrier()
        pltpu.sync_copy(
            agg_sh.at[pl.ds(sid * OROWS, OROWS)],
            out_hbm.at[cid, pl.ds(sid * OROWS, OROWS)],
        )

        @pl.when(sid == 0)
        def _():
            pltpu.sync_copy(
                agg_sh.at[pl.ds(NS * OROWS, OTAIL)],
                out_hbm.at[cid, pl.ds(NS * OROWS, OTAIL)],
            )

    return sc_stage


def _node_body(h_ref, a_ref, e_ref, w1_ref, b1_ref, g1_ref, be1_ref,
               w2_ref, b2_ref, go_ref, bo_ref, o_ref, *, final_relu, layer):
    sl = slice(layer, layer + 1)
    agg = a_ref[0] + a_ref[1]
    z = (1.0 + e_ref[0, layer]) * h_ref[...] + agg
    t = jnp.dot(z, w1_ref[0], preferred_element_type=jnp.float32)
    t = t + b1_ref[sl, :]
    m = jnp.mean(t, axis=0, keepdims=True)
    v = jnp.mean((t - m) ** 2, axis=0, keepdims=True)
    t = (t - m) * lax.rsqrt(v + 1e-5) * g1_ref[sl, :] + be1_ref[sl, :]
    t = jnp.maximum(t, 0.0)
    u = jnp.dot(t, w2_ref[0], preferred_element_type=jnp.float32)
    u = u + b2_ref[sl, :]
    m2 = jnp.mean(u, axis=0, keepdims=True)
    v2 = jnp.mean((u - m2) ** 2, axis=0, keepdims=True)
    y = (u - m2) * lax.rsqrt(v2 + 1e-5) * go_ref[sl, :] + bo_ref[sl, :]
    if final_relu:
        y = jnp.maximum(y, 0.0)
    o_ref[...] = y


def _node_stage(layer, final_relu, h_rows, interpret=False):
    l = layer
    return pl.pallas_call(
        functools.partial(_node_body, final_relu=final_relu, layer=l),
        grid=(1,),
        in_specs=[
            pl.BlockSpec((h_rows, D), lambda i: (0, 0)),
            pl.BlockSpec((NC, N, D), lambda i: (0, 0, 0)),
            pl.BlockSpec((1, 2), lambda i: (0, 0)),
            pl.BlockSpec((1, D, HID), lambda i: (l, 0, 0)),
            pl.BlockSpec((2, HID), lambda i: (0, 0)),
            pl.BlockSpec((2, HID), lambda i: (0, 0)),
            pl.BlockSpec((2, HID), lambda i: (0, 0)),
            pl.BlockSpec((1, HID, D), lambda i: (l, 0, 0)),
            pl.BlockSpec((2, D), lambda i: (0, 0)),
            pl.BlockSpec((2, D), lambda i: (0, 0)),
            pl.BlockSpec((2, D), lambda i: (0, 0)),
        ],
        out_specs=pl.BlockSpec((N, D), lambda i: (0, 0)),
        out_shape=jax.ShapeDtypeStruct((N, D), jnp.float32),
        interpret=interpret,
    )


def kernel(x, edge_index, edge_attr, node_table, edge_W, edge_b, eps,
           W1, b1, g1, be1, W2, b2, g_out, b_out):
    f32 = jnp.float32
    ei = edge_index.reshape(2 * E)
    eps2 = eps.reshape(1, 2)

    msg0 = _edge_stage(0, relu=True)(edge_attr, edge_W, edge_b, node_table)
    e1 = _edge_stage(1, relu=False)(edge_attr, edge_W, edge_b)

    zeros = jnp.zeros((ZROWS, D), f32)
    agg0 = _make_sc_scatter()(msg0, ei, zeros)

    h1 = _node_stage(0, final_relu=True, h_rows=1)(
        node_table, agg0, eps2, W1, b1, g1, be1, W2, b2, g_out, b_out)

    agg1 = _make_sc_stage(with_gather=True)(e1, ei, zeros, h1)

    h2 = _node_stage(1, final_relu=False, h_rows=N)(
        h1, agg1, eps2, W1, b1, g1, be1, W2, b2, g_out, b_out)
    return h2


# speedup vs baseline: 1.4574x; 1.0010x over previous
import functools

import jax
import jax.numpy as jnp
from jax import lax
from jax.experimental import pallas as pl
from jax.experimental.pallas import tpu as pltpu
from jax.experimental.pallas import tpu_sc as plsc

N = 10000
E = 320000
D = 128
HID = 2 * D
EDGE_DIM = 7

NC = 2
NS = 16
NW = NC * NS
EPW = E // NW
CH = 80
NCH = EPW // CH
N_PAD = 10240
ZROWS = N_PAD // NS
OROWS = 624
OTAIL = N - NS * OROWS

BE = 3200
BS = 2560


def _split_body(ei_ref, s_ref, d_ref):
    s_ref[...] = ei_ref[0]
    d_ref[...] = ei_ref[1]


def _split_stage(interpret=False):
    return pl.pallas_call(
        _split_body,
        out_shape=[
            jax.ShapeDtypeStruct((E,), jnp.int32),
            jax.ShapeDtypeStruct((E,), jnp.int32),
        ],
        interpret=interpret,
    )


def _edge_body(*refs, relu, layer):
    if relu:
        ea_ref, w_ref, b_ref, c_ref, o_ref = refs
        bias = b_ref[layer:layer + 1, :] + c_ref[...]
    else:
        ea_ref, w_ref, b_ref, o_ref = refs
        bias = b_ref[layer:layer + 1, :]
    y = jnp.dot(ea_ref[...], w_ref[0], preferred_element_type=jnp.float32)
    y = y + bias
    if relu:
        y = jnp.maximum(y, 0.0)
    o_ref[...] = y


def _edge_stage(layer, relu, interpret=False):
    in_specs = [
        pl.BlockSpec((BE, EDGE_DIM), lambda i: (i, 0)),
        pl.BlockSpec((1, EDGE_DIM, D), lambda i: (layer, 0, 0)),
        pl.BlockSpec((2, D), lambda i: (0, 0)),
    ]
    if relu:
        in_specs.append(pl.BlockSpec((1, D), lambda i: (0, 0)))
    return pl.pallas_call(
        functools.partial(_edge_body, relu=relu, layer=layer),
        grid=(E // BE,),
        in_specs=in_specs,
        out_specs=pl.BlockSpec((BE, D), lambda i: (i, 0)),
        out_shape=jax.ShapeDtypeStruct((E, D), jnp.float32),
        interpret=interpret,
    )


RING = 4
AHEAD = 2


def _make_sc_scatter(interpret=False):
    scratch = (
        [pltpu.VMEM((CH,), jnp.int32) for _ in range(RING)]
        + [pltpu.VMEM((CH, D), jnp.float32) for _ in range(RING)]
        + [pltpu.VMEM_SHARED((N_PAD, D), jnp.float32)]
        + [pltpu.SemaphoreType.DMA for _ in range(3 * RING)]
    )
    mesh = plsc.VectorSubcoreMesh(core_axis_name="c", subcore_axis_name="s")

    @functools.partial(
        pl.kernel,
        out_type=jax.ShapeDtypeStruct((NC, N, D), jnp.float32),
        mesh=mesh,
        scratch_types=scratch,
        interpret=interpret,
    )
    def sc_stage(emb_hbm, dst_hbm, z_hbm, out_hbm, *rest):
        dsti = rest[:RING]
        ebuf = rest[RING:2 * RING]
        agg_sh = rest[2 * RING]
        dsem = rest[2 * RING + 1:3 * RING + 1]
        esem = rest[3 * RING + 1:4 * RING + 1]
        ssem = rest[4 * RING + 1:]

        cid = lax.axis_index("c")
        sid = lax.axis_index("s")
        wid = cid * NS + sid
        base = wid * EPW

        def load(j, u):
            eb = base + j * CH
            pltpu.async_copy(dst_hbm.at[pl.ds(eb, CH)], dsti[u], dsem[u])
            pltpu.async_copy(emb_hbm.at[pl.ds(eb, CH)], ebuf[u], esem[u])

        def wait_load(u):
            pltpu.make_async_copy(
                dst_hbm.at[pl.ds(0, CH)], dsti[u], dsem[u]).wait()
            pltpu.make_async_copy(
                emb_hbm.at[pl.ds(0, CH)], ebuf[u], esem[u]).wait()

        def scat(u):
            pltpu.async_copy(ebuf[u], agg_sh.at[dsti[u]], ssem[u], add=True)

        def wait_scat(u):
            pltpu.make_async_copy(
                ebuf[u], agg_sh.at[dsti[u]], ssem[u]).wait()

        pltpu.sync_copy(z_hbm, agg_sh.at[pl.ds(sid * ZROWS, ZROWS)])
        plsc.subcore_barrier()

        for j in range(AHEAD):
            load(j, j)
        for j in range(RING):
            wait_load(j)
            scat(j)
            if j >= AHEAD:
                wait_scat(j - AHEAD)
            load(j + AHEAD, (j + AHEAD) % RING)

        loop_end = ((NCH - AHEAD) // RING) * RING

        @pl.loop(RING, loop_end, step=RING)
        def _(jb):
            for u in range(RING):
                j = jb + u
                wait_load(u)
                scat(u)
                wait_scat((u + AHEAD) % RING)
                load(j + AHEAD, (u + AHEAD) % RING)

        for j in range(loop_end, NCH):
            u = j % RING
            wait_load(u)
            scat(u)
            wait_scat((u + AHEAD) % RING)
            if j + AHEAD < NCH:
                load(j + AHEAD, (j + AHEAD) % RING)
        for j in range(NCH - AHEAD, NCH):
            wait_scat(j % RING)

        plsc.subcore_barrier()
        pltpu.sync_copy(
            agg_sh.at[pl.ds(sid * OROWS, OROWS)],
            out_hbm.at[cid, pl.ds(sid * OROWS, OROWS)],
        )

        @pl.when(sid == 0)
        def _():
            pltpu.sync_copy(
                agg_sh.at[pl.ds(NS * OROWS, OTAIL)],
                out_hbm.at[cid, pl.ds(NS * OROWS, OTAIL)],
            )

    return sc_stage


def _make_sc_stage(with_gather, interpret=False):
    scratch = [
        pltpu.VMEM((CH,), jnp.int32),
        pltpu.VMEM((CH,), jnp.int32),
        pltpu.VMEM((CH, D), jnp.float32),
        pltpu.VMEM((CH, D), jnp.float32),
        pltpu.VMEM_SHARED((N_PAD, D), jnp.float32),
        pltpu.SemaphoreType.DMA,
        pltpu.SemaphoreType.DMA,
        pltpu.SemaphoreType.DMA,
        pltpu.SemaphoreType.DMA,
    ]
    if with_gather:
        scratch += [
            pltpu.VMEM((CH,), jnp.int32),
            pltpu.VMEM((CH,), jnp.int32),
            pltpu.VMEM((CH, D), jnp.float32),
            pltpu.VMEM((CH, D), jnp.float32),
            pltpu.SemaphoreType.DMA,
            pltpu.SemaphoreType.DMA,
            pltpu.SemaphoreType.DMA,
            pltpu.SemaphoreType.DMA,
        ]
    mesh = plsc.VectorSubcoreMesh(core_axis_name="c", subcore_axis_name="s")

    @functools.partial(
        pl.kernel,
        out_type=jax.ShapeDtypeStruct((NC, N, D), jnp.float32),
        mesh=mesh,
        scratch_types=scratch,
        interpret=interpret,
    )
    def sc_stage(*args):
        if with_gather:
            (emb_hbm, src_hbm, dst_hbm, z_hbm, h_hbm, out_hbm,
             dsti0, dsti1, ebuf0, ebuf1, agg_sh, sd0, sd1, se0, se1,
             srci0, srci1, rows0, rows1, ss0, ss1, sg0, sg1) = args
            srci = (srci0, srci1)
            rows = (rows0, rows1)
            ssem = (ss0, ss1)
            gsem = (sg0, sg1)
        else:
            (emb_hbm, src_hbm, dst_hbm, z_hbm, out_hbm,
             dsti0, dsti1, ebuf0, ebuf1, agg_sh, sd0, sd1, se0, se1) = args
        dsti = (dsti0, dsti1)
        ebuf = (ebuf0, ebuf1)
        dsem = (sd0, sd1)
        esem = (se0, se1)

        cid = lax.axis_index("c")
        sid = lax.axis_index("s")
        wid = cid * NS + sid
        base = wid * EPW

        def start_loads(j, s):
            eb = base + j * CH
            pltpu.make_async_copy(
                dst_hbm.at[pl.ds(eb, CH)], dsti[s], dsem[s]).start()
            pltpu.make_async_copy(
                emb_hbm.at[pl.ds(eb, CH)], ebuf[s], esem[s]).start()
            if with_gather:
                pltpu.make_async_copy(
                    src_hbm.at[pl.ds(eb, CH)], srci[s], ssem[s]).start()

        def wait_loads(s):
            pltpu.make_async_copy(
                dst_hbm.at[pl.ds(0, CH)], dsti[s], dsem[s]).wait()
            pltpu.make_async_copy(
                emb_hbm.at[pl.ds(0, CH)], ebuf[s], esem[s]).wait()

        def start_gather(s):
            pltpu.make_async_copy(
                src_hbm.at[pl.ds(0, CH)], srci[s], ssem[s]).wait()
            pltpu.make_async_copy(h_hbm.at[srci[s]], rows[s], gsem[s]).start()

        def process(s):
            wait_loads(s)
            if with_gather:
                pltpu.make_async_copy(
                    h_hbm.at[srci[s]], rows[s], gsem[s]).wait()

                @pl.loop(0, CH, step=2)
                def _(r):
                    for rr in range(2):
                        for q in range(D // 16):
                            sl = pl.ds(q * 16, 16)
                            v = (ebuf[s].at[r + rr, sl][...]
                                 + rows[s].at[r + rr, sl][...])
                            ebuf[s].at[r + rr, sl][...] = jnp.maximum(v, 0.0)

                pltpu.sync_copy(ebuf[s], agg_sh.at[dsti[s]], add=True)
            else:
                pltpu.sync_copy(ebuf[s], agg_sh.at[dsti[s]], add=True)

        pltpu.sync_copy(z_hbm, agg_sh.at[pl.ds(sid * ZROWS, ZROWS)])
        plsc.subcore_barrier()

        start_loads(0, 0)
        start_loads(1, 1)
        if with_gather:
            start_gather(0)

        @pl.loop(0, (NCH - 1) // 2)
        def _(k):
            j = 2 * k
            process(0)
            start_loads(j + 2, 0)
            if with_gather:
                start_gather(1)
            process(1)

            @pl.when(j + 3 < NCH)
            def _():
                start_loads(j + 3, 1)

            if with_gather:
                start_gather(0)

        process(0)

        plsc.subcore_barrier()
        pltpu.sync_copy(
            agg_sh.at[pl.ds(sid * OROWS, OROWS)],
            out_hbm.at[cid, pl.ds(sid * OROWS, OROWS)],
        )

        @pl.when(sid == 0)
        def _():
            pltpu.sync_copy(
                agg_sh.at[pl.ds(NS * OROWS, OTAIL)],
                out_hbm.at[cid, pl.ds(NS * OROWS, OTAIL)],
            )

    return sc_stage


def _node_body(h_ref, a_ref, e_ref, w1_ref, b1_ref, g1_ref, be1_ref,
               w2_ref, b2_ref, go_ref, bo_ref, o_ref, *, final_relu, layer):
    sl = slice(layer, layer + 1)
    agg = a_ref[0] + a_ref[1]
    z = (1.0 + e_ref[0, layer]) * h_ref[...] + agg
    t = jnp.dot(z, w1_ref[0], preferred_element_type=jnp.float32)
    t = t + b1_ref[sl, :]
    m = jnp.mean(t, axis=0, keepdims=True)
    v = jnp.mean((t - m) ** 2, axis=0, keepdims=True)
    t = (t - m) * lax.rsqrt(v + 1e-5) * g1_ref[sl, :] + be1_ref[sl, :]
    t = jnp.maximum(t, 0.0)
    u = jnp.dot(t, w2_ref[0], preferred_element_type=jnp.float32)
    u = u + b2_ref[sl, :]
    m2 = jnp.mean(u, axis=0, keepdims=True)
    v2 = jnp.mean((u - m2) ** 2, axis=0, keepdims=True)
    y = (u - m2) * lax.rsqrt(v2 + 1e-5) * go_ref[sl, :] + bo_ref[sl, :]
    if final_relu:
        y = jnp.maximum(y, 0.0)
    o_ref[...] = y


def _node_stage(layer, final_relu, h_rows, interpret=False):
    l = layer
    return pl.pallas_call(
        functools.partial(_node_body, final_relu=final_relu, layer=l),
        grid=(1,),
        in_specs=[
            pl.BlockSpec((h_rows, D), lambda i: (0, 0)),
            pl.BlockSpec((NC, N, D), lambda i: (0, 0, 0)),
            pl.BlockSpec((1, 2), lambda i: (0, 0)),
            pl.BlockSpec((1, D, HID), lambda i: (l, 0, 0)),
            pl.BlockSpec((2, HID), lambda i: (0, 0)),
            pl.BlockSpec((2, HID), lambda i: (0, 0)),
            pl.BlockSpec((2, HID), lambda i: (0, 0)),
            pl.BlockSpec((1, HID, D), lambda i: (l, 0, 0)),
            pl.BlockSpec((2, D), lambda i: (0, 0)),
            pl.BlockSpec((2, D), lambda i: (0, 0)),
            pl.BlockSpec((2, D), lambda i: (0, 0)),
        ],
        out_specs=pl.BlockSpec((N, D), lambda i: (0, 0)),
        out_shape=jax.ShapeDtypeStruct((N, D), jnp.float32),
        interpret=interpret,
    )


def kernel(x, edge_index, edge_attr, node_table, edge_W, edge_b, eps,
           W1, b1, g1, be1, W2, b2, g_out, b_out):
    f32 = jnp.float32
    src, dst = _split_stage()(edge_index)
    eps2 = eps.reshape(1, 2)

    msg0 = _edge_stage(0, relu=True)(edge_attr, edge_W, edge_b, node_table)
    e1 = _edge_stage(1, relu=False)(edge_attr, edge_W, edge_b)

    zeros = jnp.zeros((ZROWS, D), f32)
    agg0 = _make_sc_scatter()(msg0, dst, zeros)

    h1 = _node_stage(0, final_relu=True, h_rows=1)(
        node_table, agg0, eps2, W1, b1, g1, be1, W2, b2, g_out, b_out)

    agg1 = _make_sc_stage(with_gather=True)(e1, src, dst, zeros, h1)

    h2 = _node_stage(1, final_relu=False, h_rows=N)(
        h1, agg1, eps2, W1, b1, g1, be1, W2, b2, g_out, b_out)
    return h2
